# Initial kernel scaffold; baseline (speedup 1.0000x reference)
import jax
import jax.numpy as jnp
from jax.experimental import pallas as pl


def kernel(x, gaze):
    raise NotImplementedError("write your pallas kernel here")


# speedup vs baseline: 4.6839x; 4.6839x over previous
import jax
import jax.numpy as jnp
from jax.experimental import pallas as pl

_K = 16
_ROUNDS = 31
_BLK = 256


def _knn_body(xr_ref, xc_ref, gr_ref, gc_ref, nn_ref):
    xr = xr_ref[0]
    xc = xc_ref[0]
    gr = gr_ref[0]
    gc = gc_ref[0]

    rden = jnp.clip(jnp.sqrt(jnp.sum(xr * xr, axis=1, keepdims=True)), 1e-12, None)
    xrn = xr / rden
    cden = jnp.clip(jnp.sqrt(jnp.sum(xc * xc, axis=0, keepdims=True)), 1e-12, None)
    xcn = xc / cden

    xsq_r = jnp.sum(xrn * xrn, axis=1, keepdims=True)
    xsq_c = jnp.sum(xcn * xcn, axis=0, keepdims=True)

    inner = jax.lax.dot_general(
        xrn, xcn, (((1,), (0,)), ((), ())),
        preferred_element_type=jnp.float32,
        precision=jax.lax.Precision.DEFAULT,
    )
    dist = (xsq_r + (-2.0 * inner)) + xsq_c

    gmax = jnp.max(gc)
    gmin = jnp.min(gc)
    gd = (gr * gr + (-2.0 * (gr * gc))) + gc * gc
    gnorm_r = (gr - gmin) / (gmax - gmin)
    dist = dist + gd * gnorm_r

    n = dist.shape[1]
    colio = jax.lax.broadcasted_iota(jnp.int32, dist.shape, 1)
    cols = []
    for r in range(_ROUNDS):
        m = jnp.min(dist, axis=1, keepdims=True)
        idx = jnp.min(jnp.where(dist == m, colio, jnp.int32(n)),
                      axis=1, keepdims=True)
        if r % 2 == 0:
            cols.append(idx)
        dist = jnp.where(colio == idx, jnp.float32(jnp.inf), dist)
    nn_ref[0] = jnp.concatenate(cols, axis=1)


def kernel(x, gaze):
    B, C, N, _ = x.shape
    x2 = x[..., 0]
    xt = jnp.swapaxes(x2, 1, 2)
    g = gaze[:, 0, :, 0]
    g_rows = g[:, :, None]
    g_cols = g[:, None, :]

    nn = pl.pallas_call(
        _knn_body,
        grid=(B, N // _BLK),
        in_specs=[
            pl.BlockSpec((1, _BLK, C), lambda b, i: (b, i, 0)),
            pl.BlockSpec((1, C, N), lambda b, i: (b, 0, 0)),
            pl.BlockSpec((1, _BLK, 1), lambda b, i: (b, i, 0)),
            pl.BlockSpec((1, 1, N), lambda b, i: (b, 0, 0)),
        ],
        out_specs=pl.BlockSpec((1, _BLK, _K), lambda b, i: (b, i, 0)),
        out_shape=jax.ShapeDtypeStruct((B, N, _K), jnp.int32),
    )(xt, x2, g_rows, g_cols)

    center = jnp.broadcast_to(
        jnp.arange(N, dtype=nn.dtype)[None, :, None], (B, N, _K))
    return jnp.stack((nn, center), axis=0)


# speedup vs baseline: 7.3464x; 1.5684x over previous
import jax
import jax.numpy as jnp
from jax.experimental import pallas as pl

_K = 16
_ROUNDS = 31
_BLK = 256


def _knn_body(xr_ref, xc_ref, gr_ref, gc_ref, nn_ref):
    xr = xr_ref[0]
    xc = xc_ref[0]
    gr = gr_ref[0]
    gc = gc_ref[0]

    rden = jnp.clip(jnp.sqrt(jnp.sum(xr * xr, axis=1, keepdims=True)), 1e-12, None)
    xrn = xr / rden
    cden = jnp.clip(jnp.sqrt(jnp.sum(xc * xc, axis=0, keepdims=True)), 1e-12, None)
    xcn = xc / cden

    xsq_r = jnp.sum(xrn * xrn, axis=1, keepdims=True)
    xsq_c = jnp.sum(xcn * xcn, axis=0, keepdims=True)

    inner = jax.lax.dot_general(
        xrn, xcn, (((1,), (0,)), ((), ())),
        preferred_element_type=jnp.float32,
        precision=jax.lax.Precision.DEFAULT,
    )
    dist = (xsq_r + (-2.0 * inner)) + xsq_c

    gmax = jnp.max(gc)
    gmin = jnp.min(gc)
    gd = (gr * gr + (-2.0 * (gr * gc))) + gc * gc
    gnorm_r = (gr - gmin) / (gmax - gmin)
    dist = dist + gd * gnorm_r

    n = dist.shape[1]
    colio = jax.lax.broadcasted_iota(jnp.int32, dist.shape, 1)
    cols = []
    for r in range(_ROUNDS):
        m = jnp.min(dist, axis=1, keepdims=True)
        hit = dist == m
        if r % 2 == 0:
            idx = jnp.min(jnp.where(hit, colio, jnp.int32(n)),
                          axis=1, keepdims=True)
            cols.append(idx)
        if r < _ROUNDS - 1:
            dist = jnp.where(hit, jnp.float32(jnp.inf), dist)
    nn_ref[0] = jnp.concatenate(cols, axis=1)


def kernel(x, gaze):
    B, C, N, _ = x.shape
    x2 = x[..., 0]
    xt = jnp.swapaxes(x2, 1, 2)
    g = gaze[:, 0, :, 0]
    g_rows = g[:, :, None]
    g_cols = g[:, None, :]

    nn = pl.pallas_call(
        _knn_body,
        grid=(B, N // _BLK),
        in_specs=[
            pl.BlockSpec((1, _BLK, C), lambda b, i: (b, i, 0)),
            pl.BlockSpec((1, C, N), lambda b, i: (b, 0, 0)),
            pl.BlockSpec((1, _BLK, 1), lambda b, i: (b, i, 0)),
            pl.BlockSpec((1, 1, N), lambda b, i: (b, 0, 0)),
        ],
        out_specs=pl.BlockSpec((1, _BLK, _K), lambda b, i: (b, i, 0)),
        out_shape=jax.ShapeDtypeStruct((B, N, _K), jnp.int32),
    )(xt, x2, g_rows, g_cols)

    center = jnp.broadcast_to(
        jnp.arange(N, dtype=nn.dtype)[None, :, None], (B, N, _K))
    return jnp.stack((nn, center), axis=0)
